# Initial kernel scaffold; baseline (speedup 1.0000x reference)
import jax
import jax.numpy as jnp
from jax.experimental import pallas as pl


def kernel(x, edge_index, edge_weight, emb, t_w0, t_b0, c_w0, c_b0, gnc_a0, gnc_g0, gnc_b0, t_w1, t_b1, c_w1, c_b1, gnc_a1, gnc_g1, gnc_b1, gn_a, gn_g, gn_b):
    raise NotImplementedError("write your pallas kernel here")


# speedup vs baseline: 3.0426x; 3.0426x over previous
import functools

import jax
import jax.numpy as jnp
from jax import lax
from jax.experimental import pallas as pl
from jax.experimental.pallas import tpu as pltpu
from jax.experimental.pallas import tpu_sc as plsc

N = 10000
NP = 10240
D = 128
E = 320000
EP = 327680
CH = 128
EPS = 1e-5
R = 512
GRID = NP // R

_f32 = jnp.float32
_mesh = plsc.VectorSubcoreMesh(core_axis_name="c", subcore_axis_name="s")


@functools.partial(
    pl.kernel,
    out_type=(
        jax.ShapeDtypeStruct((EP,), _f32),
        jax.ShapeDtypeStruct((NP, D), _f32),
        jax.ShapeDtypeStruct((NP,), _f32),
    ),
    mesh=_mesh,
    scratch_types=[
        pltpu.VMEM((CH,), jnp.int32),
        pltpu.VMEM((CH,), _f32),
        pltpu.VMEM((CH,), _f32),
        pltpu.VMEM((CH, D), _f32),
        pltpu.VMEM((640,), _f32),
        pltpu.VMEM_SHARED((NP,), _f32),
        pltpu.SemaphoreType.DMA,
    ],
)
def _sc_prep(src_h, ew_h, xidx_h, emb_h, wn_h, h0_h, ideg_h,
             idxv, ewv, valv, rowv, degv, deg_acc, sem):
    cid = lax.axis_index("c")
    sid = lax.axis_index("s")

    @pl.when(cid == 1)
    def _():
        base = sid * 640
        for k in range(5):
            pltpu.sync_copy(xidx_h.at[pl.ds(base + k * CH, CH)], idxv)
            pltpu.async_copy(emb_h.at[idxv], rowv, sem).wait()
            pltpu.sync_copy(rowv, h0_h.at[pl.ds(base + k * CH, CH)])

    @pl.when(cid == 0)
    def _():
        for k in range(40):
            degv[pl.ds(k * 16, 16)] = jnp.zeros((16,), _f32)
        pltpu.sync_copy(degv, deg_acc.at[pl.ds(sid * 640, 640)])
        plsc.subcore_barrier()

        ebase = sid * (EP // 16)

        def _deg_body(i, c):
            off = ebase + i * CH
            pltpu.sync_copy(src_h.at[pl.ds(off, CH)], idxv)
            pltpu.sync_copy(ew_h.at[pl.ds(off, CH)], ewv)
            pltpu.sync_copy(ewv, deg_acc.at[idxv], add=True)
            return c

        lax.fori_loop(0, EP // 16 // CH, _deg_body, 0)
        plsc.subcore_barrier()

        pltpu.sync_copy(deg_acc.at[pl.ds(sid * 640, 640)], degv)
        for k in range(40):
            dv = degv[pl.ds(k * 16, 16)]
            dv = jnp.where(dv < 0.5, dv + 1.0, dv)
            degv[pl.ds(k * 16, 16)] = 1.0 / dv
        pltpu.sync_copy(degv, ideg_h.at[pl.ds(sid * 640, 640)])
        plsc.subcore_barrier()

        def _wn_body(i, c):
            off = ebase + i * CH
            pltpu.sync_copy(src_h.at[pl.ds(off, CH)], idxv)
            pltpu.sync_copy(ew_h.at[pl.ds(off, CH)], ewv)
            pltpu.async_copy(ideg_h.at[idxv], valv, sem).wait()
            for k in range(8):
                valv[pl.ds(k * 16, 16)] = (
                    valv[pl.ds(k * 16, 16)] * ewv[pl.ds(k * 16, 16)])
            pltpu.sync_copy(valv, wn_h.at[pl.ds(off, CH)])
            return c

        lax.fori_loop(0, EP // 16 // CH, _wn_body, 0)


@functools.partial(
    pl.kernel,
    out_type=jax.ShapeDtypeStruct((2, NP, D), _f32),
    mesh=_mesh,
    scratch_types=[
        pltpu.VMEM((CH,), jnp.int32),
        pltpu.VMEM((CH,), jnp.int32),
        pltpu.VMEM((CH,), _f32),
        pltpu.VMEM((CH, D), _f32),
        pltpu.VMEM_SHARED((NP, D), _f32),
        pltpu.SemaphoreType.DMA,
    ],
)
def _sc_spmm(dst_h, src_h, wn_h, a_h, out_h,
             dstv, srcv, wv, rowv, acc, sem):
    cid = lax.axis_index("c")
    sid = lax.axis_index("s")

    def _z(i, c):
        for g in range(8):
            rowv[i, pl.ds(g * 16, 16)] = jnp.zeros((16,), _f32)
        return c

    lax.fori_loop(0, CH, _z, 0)
    nbase = sid * 640
    for k in range(5):
        pltpu.sync_copy(rowv, acc.at[pl.ds(nbase + k * CH, CH)])
    plsc.subcore_barrier()

    ebase = cid * (EP // 2) + sid * (EP // 32)

    def _body(i, c):
        off = ebase + i * CH
        pltpu.sync_copy(dst_h.at[pl.ds(off, CH)], dstv)
        pltpu.sync_copy(src_h.at[pl.ds(off, CH)], srcv)
        pltpu.sync_copy(wn_h.at[pl.ds(off, CH)], wv)
        pltpu.async_copy(a_h.at[dstv], rowv, sem).wait()

        def _scale(g, cc):
            wgrp = wv[pl.ds(g * 16, 16)]
            for lane in range(16):
                wvec = jnp.full((16,), wgrp[lane], _f32)
                e = g * 16 + lane
                for cg in range(8):
                    rowv[e, pl.ds(cg * 16, 16)] = (
                        rowv[e, pl.ds(cg * 16, 16)] * wvec)
            return cc

        lax.fori_loop(0, CH // 16, _scale, 0)
        pltpu.sync_copy(rowv, acc.at[srcv], add=True)
        return c

    lax.fori_loop(0, EP // 32 // CH, _body, 0)
    plsc.subcore_barrier()

    for k in range(5):
        pltpu.sync_copy(acc.at[pl.ds(nbase + k * CH, CH)],
                        out_h.at[cid, pl.ds(nbase + k * CH, CH)])


def _mm_relu_body(x_ref, w_ref, b_ref, o_ref):
    o_ref[...] = jnp.maximum(
        jnp.dot(x_ref[...], w_ref[...], preferred_element_type=_f32)
        + b_ref[...], 0.0)


def _mm_relu(x, w, b):
    return pl.pallas_call(
        _mm_relu_body,
        grid=(GRID,),
        in_specs=[
            pl.BlockSpec((R, D), lambda i: (i, 0)),
            pl.BlockSpec((D, D), lambda i: (0, 0)),
            pl.BlockSpec((1, D), lambda i: (0, 0)),
        ],
        out_specs=pl.BlockSpec((R, D), lambda i: (i, 0)),
        out_shape=jax.ShapeDtypeStruct((NP, D), _f32),
    )(x, w, b)


def _stats_body(p0_ref, p1_ref, o_ref):
    i = pl.program_id(0)
    s = p0_ref[...] + p1_ref[...]
    s1 = jnp.sum(s, axis=0, keepdims=True)
    s2 = jnp.sum(s * s, axis=0, keepdims=True)
    st = jnp.concatenate([s1, s2, jnp.zeros((6, D), _f32)], axis=0)

    @pl.when(i == 0)
    def _():
        o_ref[...] = st

    @pl.when(i > 0)
    def _():
        o_ref[...] = o_ref[...] + st


def _stats(p0, p1):
    return pl.pallas_call(
        _stats_body,
        grid=(GRID,),
        in_specs=[
            pl.BlockSpec((R, D), lambda i: (i, 0)),
            pl.BlockSpec((R, D), lambda i: (i, 0)),
        ],
        out_specs=pl.BlockSpec((8, D), lambda i: (0, 0)),
        out_shape=jax.ShapeDtypeStruct((8, D), _f32),
    )(p0, p1)


def _gn_from_stats(s, st_ref, ga, gg, gb):
    m = st_ref[0:1, :] * (1.0 / N)
    ex2 = st_ref[1:2, :] * (1.0 / N)
    var = ex2 - (2.0 * ga - ga * ga) * m * m
    sub = s - ga * m
    return gg * sub * lax.rsqrt(var + EPS) + gb


def _apply0_body(p0_ref, p1_ref, st_ref, h0_ref, ga_ref, gg_ref, gb_ref,
                 cwa_ref, cwb_ref, cb_ref, h1_ref, st1_ref):
    i = pl.program_id(0)
    s = p0_ref[...] + p1_ref[...]
    y = _gn_from_stats(s, st_ref, ga_ref[...], gg_ref[...], gb_ref[...])
    h1 = (jnp.dot(y, cwa_ref[...], preferred_element_type=_f32)
          + jnp.dot(h0_ref[...], cwb_ref[...], preferred_element_type=_f32)
          + cb_ref[...])
    rows = lax.broadcasted_iota(jnp.int32, (R, D), 0) + i * R
    h1 = jnp.where(rows < N, h1, 0.0)
    h1_ref[...] = h1
    s1 = jnp.sum(h1, axis=0, keepdims=True)
    s2 = jnp.sum(h1 * h1, axis=0, keepdims=True)
    st = jnp.concatenate([s1, s2, jnp.zeros((6, D), _f32)], axis=0)

    @pl.when(i == 0)
    def _():
        st1_ref[...] = st

    @pl.when(i > 0)
    def _():
        st1_ref[...] = st1_ref[...] + st


def _apply0(p0, p1, st, h0, ga, gg, gb, cwa, cwb, cb):
    return pl.pallas_call(
        _apply0_body,
        grid=(GRID,),
        in_specs=[
            pl.BlockSpec((R, D), lambda i: (i, 0)),
            pl.BlockSpec((R, D), lambda i: (i, 0)),
            pl.BlockSpec((8, D), lambda i: (0, 0)),
            pl.BlockSpec((R, D), lambda i: (i, 0)),
            pl.BlockSpec((1, D), lambda i: (0, 0)),
            pl.BlockSpec((1, D), lambda i: (0, 0)),
            pl.BlockSpec((1, D), lambda i: (0, 0)),
            pl.BlockSpec((D, D), lambda i: (0, 0)),
            pl.BlockSpec((D, D), lambda i: (0, 0)),
            pl.BlockSpec((1, D), lambda i: (0, 0)),
        ],
        out_specs=[
            pl.BlockSpec((R, D), lambda i: (i, 0)),
            pl.BlockSpec((8, D), lambda i: (0, 0)),
        ],
        out_shape=[
            jax.ShapeDtypeStruct((NP, D), _f32),
            jax.ShapeDtypeStruct((8, D), _f32),
        ],
    )(p0, p1, st, h0, ga, gg, gb, cwa, cwb, cb)


def _mid_body(h1_ref, st_ref, ga_ref, gg_ref, gb_ref, tw_ref, tb_ref,
              h_ref, a1_ref):
    y = _gn_from_stats(h1_ref[...], st_ref, ga_ref[...], gg_ref[...],
                       gb_ref[...])
    h = jnp.maximum(y, 0.0)
    h_ref[...] = h
    a1_ref[...] = jnp.maximum(
        jnp.dot(h, tw_ref[...], preferred_element_type=_f32) + tb_ref[...],
        0.0)


def _mid(h1, st, ga, gg, gb, tw, tb):
    return pl.pallas_call(
        _mid_body,
        grid=(GRID,),
        in_specs=[
            pl.BlockSpec((R, D), lambda i: (i, 0)),
            pl.BlockSpec((8, D), lambda i: (0, 0)),
            pl.BlockSpec((1, D), lambda i: (0, 0)),
            pl.BlockSpec((1, D), lambda i: (0, 0)),
            pl.BlockSpec((1, D), lambda i: (0, 0)),
            pl.BlockSpec((D, D), lambda i: (0, 0)),
            pl.BlockSpec((1, D), lambda i: (0, 0)),
        ],
        out_specs=[
            pl.BlockSpec((R, D), lambda i: (i, 0)),
            pl.BlockSpec((R, D), lambda i: (i, 0)),
        ],
        out_shape=[
            jax.ShapeDtypeStruct((NP, D), _f32),
            jax.ShapeDtypeStruct((NP, D), _f32),
        ],
    )(h1, st, ga, gg, gb, tw, tb)


def _apply1_body(p0_ref, p1_ref, st_ref, h_ref, ga_ref, gg_ref, gb_ref,
                 cwa_ref, cwb_ref, cb_ref, o_ref):
    s = p0_ref[...] + p1_ref[...]
    y = _gn_from_stats(s, st_ref, ga_ref[...], gg_ref[...], gb_ref[...])
    o_ref[...] = (jnp.dot(y, cwa_ref[...], preferred_element_type=_f32)
                  + jnp.dot(h_ref[...], cwb_ref[...],
                            preferred_element_type=_f32)
                  + cb_ref[...])


def _apply1(p0, p1, st, h, ga, gg, gb, cwa, cwb, cb):
    return pl.pallas_call(
        _apply1_body,
        grid=(GRID,),
        in_specs=[
            pl.BlockSpec((R, D), lambda i: (i, 0)),
            pl.BlockSpec((R, D), lambda i: (i, 0)),
            pl.BlockSpec((8, D), lambda i: (0, 0)),
            pl.BlockSpec((R, D), lambda i: (i, 0)),
            pl.BlockSpec((1, D), lambda i: (0, 0)),
            pl.BlockSpec((1, D), lambda i: (0, 0)),
            pl.BlockSpec((1, D), lambda i: (0, 0)),
            pl.BlockSpec((D, D), lambda i: (0, 0)),
            pl.BlockSpec((D, D), lambda i: (0, 0)),
            pl.BlockSpec((1, D), lambda i: (0, 0)),
        ],
        out_specs=pl.BlockSpec((R, D), lambda i: (i, 0)),
        out_shape=jax.ShapeDtypeStruct((NP, D), _f32),
    )(p0, p1, st, h, ga, gg, gb, cwa, cwb, cb)


def kernel(x, edge_index, edge_weight, emb,
           t_w0, t_b0, c_w0, c_b0, gnc_a0, gnc_g0, gnc_b0,
           t_w1, t_b1, c_w1, c_b1, gnc_a1, gnc_g1, gnc_b1,
           gn_a, gn_g, gn_b):
    src = jnp.pad(edge_index[0], (0, EP - E))
    dst = jnp.pad(edge_index[1], (0, EP - E))
    ew = jnp.pad(edge_weight, (0, EP - E))
    xp = jnp.pad(x.astype(jnp.int32), (0, NP - N))

    w_norm, h0, _ = _sc_prep(src, ew, xp, emb)

    def r2(v):
        return v.reshape(1, D)

    a0 = _mm_relu(h0, t_w0, r2(t_b0))
    p = _sc_spmm(dst, src, w_norm, a0)
    p0, p1 = p[0], p[1]
    st0 = _stats(p0, p1)
    h1, st1 = _apply0(p0, p1, st0, h0,
                      r2(gnc_a0), r2(gnc_g0), r2(gnc_b0),
                      c_w0[:D], c_w0[D:], r2(c_b0))
    h, a1 = _mid(h1, st1, r2(gn_a), r2(gn_g), r2(gn_b), t_w1, r2(t_b1))
    q = _sc_spmm(dst, src, w_norm, a1)
    q0, q1 = q[0], q[1]
    st2 = _stats(q0, q1)
    out = _apply1(q0, q1, st2, h,
                  r2(gnc_a1), r2(gnc_g1), r2(gnc_b1),
                  c_w1[:D], c_w1[D:], r2(c_b1))
    return out[:N]


# speedup vs baseline: 5.4124x; 1.7789x over previous
import functools

import jax
import jax.numpy as jnp
from jax import lax
from jax.experimental import pallas as pl
from jax.experimental.pallas import tpu as pltpu
from jax.experimental.pallas import tpu_sc as plsc

N = 10000
NP = 10240
D = 128
E = 320000
EP = 327680
CH = 128
EPS = 1e-5
R = 1000
GRID = N // R

CPW = EP // 32 // CH
CPT = EP // 16 // CH

_f32 = jnp.float32
_mesh = plsc.VectorSubcoreMesh(core_axis_name="c", subcore_axis_name="s")


@functools.partial(
    pl.kernel,
    out_type=(
        jax.ShapeDtypeStruct((EP // CH, CH), _f32),
        jax.ShapeDtypeStruct((NP, D), _f32),
        jax.ShapeDtypeStruct((NP,), _f32),
    ),
    mesh=_mesh,
    scratch_types=[
        pltpu.VMEM((CH,), jnp.int32),
        pltpu.VMEM((CH, D), _f32),
        pltpu.VMEM((CPT, CH), jnp.int32),
        pltpu.VMEM((CPT, CH), _f32),
        pltpu.VMEM((CPT, CH), _f32),
        pltpu.VMEM((640,), _f32),
        pltpu.VMEM_SHARED((NP,), _f32),
        pltpu.SemaphoreType.DMA,
        pltpu.SemaphoreType.DMA,
    ],
)
def _sc_prep(src_h, ew_h, xidx_h, emb_h, wn_h, h0_h, ideg_h,
             idxv, rowv, srcm, ewm, idegm, degv, deg_acc, sem, dsem):
    cid = lax.axis_index("c")
    sid = lax.axis_index("s")

    @pl.when(cid == 1)
    def _():
        base = sid * 640
        for k in range(5):
            pltpu.sync_copy(xidx_h.at[pl.ds(base + k * CH, CH)], idxv)
            pltpu.async_copy(emb_h.at[idxv], rowv, sem).wait()
            pltpu.sync_copy(rowv, h0_h.at[pl.ds(base + k * CH, CH)])

    @pl.when(cid == 0)
    def _():
        pltpu.sync_copy(src_h.at[pl.ds(sid * CPT, CPT)], srcm)
        pltpu.sync_copy(ew_h.at[pl.ds(sid * CPT, CPT)], ewm)
        for k in range(40):
            degv[pl.ds(k * 16, 16)] = jnp.zeros((16,), _f32)
        pltpu.sync_copy(degv, deg_acc.at[pl.ds(sid * 640, 640)])
        plsc.subcore_barrier()

        def _deg_body(io, c):
            for k in range(8):
                j = io * 8 + k
                pltpu.async_copy(ewm.at[j], deg_acc.at[srcm.at[j]], dsem,
                                 add=True)
            for k in range(8):
                pltpu.make_async_copy(ewm.at[0], deg_acc.at[pl.ds(0, CH)],
                                      dsem).wait()
            return c

        lax.fori_loop(0, CPT // 8, _deg_body, 0)
        plsc.subcore_barrier()

        pltpu.sync_copy(deg_acc.at[pl.ds(sid * 640, 640)], degv)
        for k in range(40):
            dv = degv[pl.ds(k * 16, 16)]
            dv = jnp.where(dv < 0.5, dv + 1.0, dv)
            degv[pl.ds(k * 16, 16)] = 1.0 / dv
        pltpu.sync_copy(degv, ideg_h.at[pl.ds(sid * 640, 640)])
        plsc.subcore_barrier()

        def _ig(io, c):
            for k in range(8):
                j = io * 8 + k
                pltpu.async_copy(ideg_h.at[srcm.at[j]], idegm.at[j], dsem)
            for k in range(8):
                pltpu.make_async_copy(ideg_h.at[pl.ds(0, CH)], idegm.at[0],
                                      dsem).wait()
            return c

        lax.fori_loop(0, CPT // 8, _ig, 0)

        def _mul(j, c):
            for g in range(8):
                ewm[j, pl.ds(g * 16, 16)] = (
                    ewm[j, pl.ds(g * 16, 16)] * idegm[j, pl.ds(g * 16, 16)])
            return c

        lax.fori_loop(0, CPT, _mul, 0)
        pltpu.sync_copy(ewm, wn_h.at[pl.ds(sid * CPT, CPT)])


NBUF = 3
NCTX = 4


@functools.partial(
    pl.kernel,
    out_type=jax.ShapeDtypeStruct((2, N, D), _f32),
    mesh=_mesh,
    scratch_types=[
        [pltpu.VMEM((CH,), jnp.int32) for _ in range(NCTX)],
        [pltpu.VMEM((CH,), jnp.int32) for _ in range(NCTX)],
        [pltpu.VMEM((CH,), _f32) for _ in range(NCTX)],
        [pltpu.VMEM((CH, D), _f32) for _ in range(NBUF)],
        pltpu.VMEM_SHARED((N, D), _f32),
        [pltpu.SemaphoreType.DMA for _ in range(NCTX)],
        [pltpu.SemaphoreType.DMA for _ in range(NBUF)],
        [pltpu.SemaphoreType.DMA for _ in range(NBUF)],
    ],
)
def _sc_spmm(dst_h, src_h, wn_h, a_h, out_h,
             dstv, srcv, wnv, rows, acc, isem, gsem, ssem):
    cid = lax.axis_index("c")
    sid = lax.axis_index("s")
    wid = cid * 16 + sid
    rbase = wid * CPW

    def _z(i, c):
        for g in range(8):
            rows[0][i, pl.ds(g * 16, 16)] = jnp.zeros((16,), _f32)
        return c

    lax.fori_loop(0, CH, _z, 0)
    nbase = sid * 640

    @pl.when(sid < 15)
    def _():
        for k in range(5):
            pltpu.sync_copy(rows[0], acc.at[pl.ds(nbase + k * CH, CH)])

    @pl.when(sid == 15)
    def _():
        for k in range(3):
            pltpu.sync_copy(rows[0], acc.at[pl.ds(nbase + k * CH, CH)])
        pltpu.sync_copy(rows[0].at[pl.ds(0, 16)],
                        acc.at[pl.ds(nbase + 3 * CH, 16)])

    plsc.subcore_barrier()

    def _issue_idx(j, c):
        pltpu.async_copy(dst_h.at[rbase + j], dstv[c], isem[c])
        pltpu.async_copy(src_h.at[rbase + j], srcv[c], isem[c])
        pltpu.async_copy(wn_h.at[rbase + j], wnv[c], isem[c])

    def _wait_idx(c):
        pltpu.make_async_copy(dst_h.at[0], dstv[c], isem[c]).wait()
        pltpu.make_async_copy(src_h.at[0], srcv[c], isem[c]).wait()
        pltpu.make_async_copy(wn_h.at[0], wnv[c], isem[c]).wait()

    def _gather(b, c):
        pltpu.async_copy(a_h.at[dstv[c]], rows[b], gsem[b])

    def _wait_gather(b):
        pltpu.make_async_copy(a_h.at[pl.ds(0, CH)], rows[b], gsem[b]).wait()

    def _scatter(b, c):
        pltpu.async_copy(rows[b], acc.at[srcv[c]], ssem[b], add=True)

    def _wait_scatter(b):
        pltpu.make_async_copy(rows[b], acc.at[pl.ds(0, CH)], ssem[b]).wait()

    def _scale(b, c):
        def _sg(g, cc):
            wgrp = wnv[c][pl.ds(g * 16, 16)]
            for lane in range(16):
                wvec = jnp.full((16,), wgrp[lane], _f32)
                e = g * 16 + lane
                for cg in range(8):
                    rows[b][e, pl.ds(cg * 16, 16)] = (
                        rows[b][e, pl.ds(cg * 16, 16)] * wvec)
            return cc

        lax.fori_loop(0, CH // 16, _sg, 0)

    def _slot(j, m3, m4, do_sw, do_idx, do_g):
        _wait_gather(m3)
        _scale(m3, m4)
        _scatter(m3, m4)
        if do_sw:
            _wait_scatter((m3 + 2) % NBUF)
        if do_idx:
            _issue_idx(j + 3, (m4 + 3) % NCTX)
        if do_g:
            _wait_idx((m4 + 2) % NCTX)
            _gather((m3 + 2) % NBUF, (m4 + 2) % NCTX)

    for c in range(3):
        _issue_idx(c, c)
    _wait_idx(0)
    _gather(0, 0)
    _wait_idx(1)
    _gather(1, 1)

    _slot(0, 0, 0, False, True, True)

    def _steady(io, c):
        for k in range(12):
            j = 1 + io * 12 + k
            _slot(j, (1 + k) % NBUF, (1 + k) % NCTX, True, True, True)
        return c

    lax.fori_loop(0, 6, _steady, 0)

    for j in range(73, CPW):
        _slot(j, j % NBUF, j % NCTX, True, j + 3 < CPW, j + 2 < CPW)

    _wait_scatter((CPW - 1) % NBUF)
    plsc.subcore_barrier()

    @pl.when(sid < 15)
    def _():
        for k in range(5):
            pltpu.sync_copy(acc.at[pl.ds(nbase + k * CH, CH)],
                            out_h.at[cid, pl.ds(nbase + k * CH, CH)])

    @pl.when(sid == 15)
    def _():
        for k in range(3):
            pltpu.sync_copy(acc.at[pl.ds(nbase + k * CH, CH)],
                            out_h.at[cid, pl.ds(nbase + k * CH, CH)])
        pltpu.sync_copy(acc.at[pl.ds(nbase + 3 * CH, 16)],
                        out_h.at[cid, pl.ds(nbase + 3 * CH, 16)])


def _mm_relu_body(x_ref, w_ref, b_ref, o_ref):
    o_ref[...] = jnp.maximum(
        jnp.dot(x_ref[...], w_ref[...], preferred_element_type=_f32)
        + b_ref[...], 0.0)


def _mm_relu(x, w, b):
    return pl.pallas_call(
        _mm_relu_body,
        grid=(GRID,),
        in_specs=[
            pl.BlockSpec((R, D), lambda i: (i, 0)),
            pl.BlockSpec((D, D), lambda i: (0, 0)),
            pl.BlockSpec((1, D), lambda i: (0, 0)),
        ],
        out_specs=pl.BlockSpec((R, D), lambda i: (i, 0)),
        out_shape=jax.ShapeDtypeStruct((N, D), _f32),
    )(x, w, b)


def _stats_body(p0_ref, p1_ref, o_ref):
    i = pl.program_id(0)
    s = p0_ref[...] + p1_ref[...]
    s1 = jnp.sum(s, axis=0, keepdims=True)
    s2 = jnp.sum(s * s, axis=0, keepdims=True)
    st = jnp.concatenate([s1, s2, jnp.zeros((6, D), _f32)], axis=0)

    @pl.when(i == 0)
    def _():
        o_ref[...] = st

    @pl.when(i > 0)
    def _():
        o_ref[...] = o_ref[...] + st


def _stats(p0, p1):
    return pl.pallas_call(
        _stats_body,
        grid=(GRID,),
        in_specs=[
            pl.BlockSpec((R, D), lambda i: (i, 0)),
            pl.BlockSpec((R, D), lambda i: (i, 0)),
        ],
        out_specs=pl.BlockSpec((8, D), lambda i: (0, 0)),
        out_shape=jax.ShapeDtypeStruct((8, D), _f32),
    )(p0, p1)


def _gn_from_stats(s, st_ref, ga, gg, gb):
    m = st_ref[0:1, :] * (1.0 / N)
    ex2 = st_ref[1:2, :] * (1.0 / N)
    var = ex2 - (2.0 * ga - ga * ga) * m * m
    sub = s - ga * m
    return gg * sub * lax.rsqrt(var + EPS) + gb


def _apply0_body(p0_ref, p1_ref, st_ref, h0_ref, ga_ref, gg_ref, gb_ref,
                 cwa_ref, cwb_ref, cb_ref, h1_ref, st1_ref):
    i = pl.program_id(0)
    s = p0_ref[...] + p1_ref[...]
    y = _gn_from_stats(s, st_ref, ga_ref[...], gg_ref[...], gb_ref[...])
    h1 = (jnp.dot(y, cwa_ref[...], preferred_element_type=_f32)
          + jnp.dot(h0_ref[...], cwb_ref[...], preferred_element_type=_f32)
          + cb_ref[...])
    h1_ref[...] = h1
    s1 = jnp.sum(h1, axis=0, keepdims=True)
    s2 = jnp.sum(h1 * h1, axis=0, keepdims=True)
    st = jnp.concatenate([s1, s2, jnp.zeros((6, D), _f32)], axis=0)

    @pl.when(i == 0)
    def _():
        st1_ref[...] = st

    @pl.when(i > 0)
    def _():
        st1_ref[...] = st1_ref[...] + st


def _apply0(p0, p1, st, h0, ga, gg, gb, cwa, cwb, cb):
    return pl.pallas_call(
        _apply0_body,
        grid=(GRID,),
        in_specs=[
            pl.BlockSpec((R, D), lambda i: (i, 0)),
            pl.BlockSpec((R, D), lambda i: (i, 0)),
            pl.BlockSpec((8, D), lambda i: (0, 0)),
            pl.BlockSpec((R, D), lambda i: (i, 0)),
            pl.BlockSpec((1, D), lambda i: (0, 0)),
            pl.BlockSpec((1, D), lambda i: (0, 0)),
            pl.BlockSpec((1, D), lambda i: (0, 0)),
            pl.BlockSpec((D, D), lambda i: (0, 0)),
            pl.BlockSpec((D, D), lambda i: (0, 0)),
            pl.BlockSpec((1, D), lambda i: (0, 0)),
        ],
        out_specs=[
            pl.BlockSpec((R, D), lambda i: (i, 0)),
            pl.BlockSpec((8, D), lambda i: (0, 0)),
        ],
        out_shape=[
            jax.ShapeDtypeStruct((N, D), _f32),
            jax.ShapeDtypeStruct((8, D), _f32),
        ],
    )(p0, p1, st, h0, ga, gg, gb, cwa, cwb, cb)


def _mid_body(h1_ref, st_ref, ga_ref, gg_ref, gb_ref, tw_ref, tb_ref,
              h_ref, a1_ref):
    y = _gn_from_stats(h1_ref[...], st_ref, ga_ref[...], gg_ref[...],
                       gb_ref[...])
    h = jnp.maximum(y, 0.0)
    h_ref[...] = h
    a1_ref[...] = jnp.maximum(
        jnp.dot(h, tw_ref[...], preferred_element_type=_f32) + tb_ref[...],
        0.0)


def _mid(h1, st, ga, gg, gb, tw, tb):
    return pl.pallas_call(
        _mid_body,
        grid=(GRID,),
        in_specs=[
            pl.BlockSpec((R, D), lambda i: (i, 0)),
            pl.BlockSpec((8, D), lambda i: (0, 0)),
            pl.BlockSpec((1, D), lambda i: (0, 0)),
            pl.BlockSpec((1, D), lambda i: (0, 0)),
            pl.BlockSpec((1, D), lambda i: (0, 0)),
            pl.BlockSpec((D, D), lambda i: (0, 0)),
            pl.BlockSpec((1, D), lambda i: (0, 0)),
        ],
        out_specs=[
            pl.BlockSpec((R, D), lambda i: (i, 0)),
            pl.BlockSpec((R, D), lambda i: (i, 0)),
        ],
        out_shape=[
            jax.ShapeDtypeStruct((N, D), _f32),
            jax.ShapeDtypeStruct((N, D), _f32),
        ],
    )(h1, st, ga, gg, gb, tw, tb)


def _apply1_body(p0_ref, p1_ref, st_ref, h_ref, ga_ref, gg_ref, gb_ref,
                 cwa_ref, cwb_ref, cb_ref, o_ref):
    s = p0_ref[...] + p1_ref[...]
    y = _gn_from_stats(s, st_ref, ga_ref[...], gg_ref[...], gb_ref[...])
    o_ref[...] = (jnp.dot(y, cwa_ref[...], preferred_element_type=_f32)
                  + jnp.dot(h_ref[...], cwb_ref[...],
                            preferred_element_type=_f32)
                  + cb_ref[...])


def _apply1(p0, p1, st, h, ga, gg, gb, cwa, cwb, cb):
    return pl.pallas_call(
        _apply1_body,
        grid=(GRID,),
        in_specs=[
            pl.BlockSpec((R, D), lambda i: (i, 0)),
            pl.BlockSpec((R, D), lambda i: (i, 0)),
            pl.BlockSpec((8, D), lambda i: (0, 0)),
            pl.BlockSpec((R, D), lambda i: (i, 0)),
            pl.BlockSpec((1, D), lambda i: (0, 0)),
            pl.BlockSpec((1, D), lambda i: (0, 0)),
            pl.BlockSpec((1, D), lambda i: (0, 0)),
            pl.BlockSpec((D, D), lambda i: (0, 0)),
            pl.BlockSpec((D, D), lambda i: (0, 0)),
            pl.BlockSpec((1, D), lambda i: (0, 0)),
        ],
        out_specs=pl.BlockSpec((R, D), lambda i: (i, 0)),
        out_shape=jax.ShapeDtypeStruct((N, D), _f32),
    )(p0, p1, st, h, ga, gg, gb, cwa, cwb, cb)


def kernel(x, edge_index, edge_weight, emb,
           t_w0, t_b0, c_w0, c_b0, gnc_a0, gnc_g0, gnc_b0,
           t_w1, t_b1, c_w1, c_b1, gnc_a1, gnc_g1, gnc_b1,
           gn_a, gn_g, gn_b):
    src = jnp.pad(edge_index[0], (0, EP - E)).reshape(EP // CH, CH)
    dst = jnp.pad(edge_index[1], (0, EP - E)).reshape(EP // CH, CH)
    ew = jnp.pad(edge_weight, (0, EP - E)).reshape(EP // CH, CH)
    xp = jnp.pad(x.astype(jnp.int32), (0, NP - N))

    wn, h0p, _ = _sc_prep(src, ew, xp, emb)
    h0 = h0p[:N]

    def r2(v):
        return v.reshape(1, D)

    a0 = _mm_relu(h0, t_w0, r2(t_b0))
    p = _sc_spmm(dst, src, wn, a0)
    p0, p1 = p[0], p[1]
    st0 = _stats(p0, p1)
    h1, st1 = _apply0(p0, p1, st0, h0,
                      r2(gnc_a0), r2(gnc_g0), r2(gnc_b0),
                      c_w0[:D], c_w0[D:], r2(c_b0))
    h, a1 = _mid(h1, st1, r2(gn_a), r2(gn_g), r2(gn_b), t_w1, r2(t_b1))
    q = _sc_spmm(dst, src, wn, a1)
    q0, q1 = q[0], q[1]
    st2 = _stats(q0, q1)
    out = _apply1(q0, q1, st2, h,
                  r2(gnc_a1), r2(gnc_g1), r2(gnc_b1),
                  c_w1[:D], c_w1[D:], r2(c_b1))
    return out


# speedup vs baseline: 14.2437x; 2.6317x over previous
import functools

import jax
import jax.numpy as jnp
from jax import lax
from jax.experimental import pallas as pl
from jax.experimental.pallas import tpu as pltpu
from jax.experimental.pallas import tpu_sc as plsc

N = 10000
NP = 10240
D = 128
E = 320000
EP = 327680
CH = 128
EPS = 1e-5
R = 1000
GRID = N // R

CPW = EP // 32 // CH
CPT = EP // 16 // CH

_f32 = jnp.float32
_mesh = plsc.VectorSubcoreMesh(core_axis_name="c", subcore_axis_name="s")


@functools.partial(
    pl.kernel,
    out_type=(
        jax.ShapeDtypeStruct((EP // CH, CH), _f32),
        jax.ShapeDtypeStruct((NP, D), _f32),
        jax.ShapeDtypeStruct((NP,), _f32),
    ),
    mesh=_mesh,
    scratch_types=[
        pltpu.VMEM((CH,), jnp.int32),
        pltpu.VMEM((CH, D), _f32),
        pltpu.VMEM((CPT, CH), jnp.int32),
        pltpu.VMEM((CPT, CH), _f32),
        pltpu.VMEM((CPT, CH), _f32),
        pltpu.VMEM((640,), _f32),
        pltpu.VMEM_SHARED((NP,), _f32),
        pltpu.SemaphoreType.DMA,
        pltpu.SemaphoreType.DMA,
    ],
)
def _sc_prep(src_h, ew_h, xidx_h, emb_h, wn_h, h0_h, ideg_h,
             idxv, rowv, srcm, ewm, idegm, degv, deg_acc, sem, dsem):
    cid = lax.axis_index("c")
    sid = lax.axis_index("s")

    @pl.when(cid == 1)
    def _():
        base = sid * 640
        for k in range(5):
            pltpu.sync_copy(xidx_h.at[pl.ds(base + k * CH, CH)], idxv)
            pltpu.async_copy(emb_h.at[idxv], rowv, sem).wait()
            pltpu.sync_copy(rowv, h0_h.at[pl.ds(base + k * CH, CH)])

    @pl.when(cid == 0)
    def _():
        pltpu.sync_copy(src_h.at[pl.ds(sid * CPT, CPT)], srcm)
        pltpu.sync_copy(ew_h.at[pl.ds(sid * CPT, CPT)], ewm)
        for k in range(40):
            degv[pl.ds(k * 16, 16)] = jnp.zeros((16,), _f32)
        pltpu.sync_copy(degv, deg_acc.at[pl.ds(sid * 640, 640)])
        plsc.subcore_barrier()

        def _deg_body(io, c):
            for k in range(8):
                j = io * 8 + k
                pltpu.async_copy(ewm.at[j], deg_acc.at[srcm.at[j]], dsem,
                                 add=True)
            for k in range(8):
                pltpu.make_async_copy(ewm.at[0], deg_acc.at[pl.ds(0, CH)],
                                      dsem).wait()
            return c

        lax.fori_loop(0, CPT // 8, _deg_body, 0)
        plsc.subcore_barrier()

        pltpu.sync_copy(deg_acc.at[pl.ds(sid * 640, 640)], degv)
        for k in range(40):
            dv = degv[pl.ds(k * 16, 16)]
            dv = jnp.where(dv < 0.5, dv + 1.0, dv)
            degv[pl.ds(k * 16, 16)] = 1.0 / dv
        pltpu.sync_copy(degv, ideg_h.at[pl.ds(sid * 640, 640)])
        plsc.subcore_barrier()

        def _ig(io, c):
            for k in range(8):
                j = io * 8 + k
                pltpu.async_copy(ideg_h.at[srcm.at[j]], idegm.at[j], dsem)
            for k in range(8):
                pltpu.make_async_copy(ideg_h.at[pl.ds(0, CH)], idegm.at[0],
                                      dsem).wait()
            return c

        lax.fori_loop(0, CPT // 8, _ig, 0)

        def _mul(j, c):
            for g in range(8):
                ewm[j, pl.ds(g * 16, 16)] = (
                    ewm[j, pl.ds(g * 16, 16)] * idegm[j, pl.ds(g * 16, 16)])
            return c

        lax.fori_loop(0, CPT, _mul, 0)
        pltpu.sync_copy(ewm, wn_h.at[pl.ds(sid * CPT, CPT)])


NBUF = 3
NCTX = 4


@functools.partial(
    pl.kernel,
    out_type=jax.ShapeDtypeStruct((2, N, D), _f32),
    mesh=_mesh,
    scratch_types=[
        [pltpu.VMEM((CH,), jnp.int32) for _ in range(NCTX)],
        [pltpu.VMEM((CH,), jnp.int32) for _ in range(NCTX)],
        [pltpu.VMEM((CH,), _f32) for _ in range(NCTX)],
        [pltpu.VMEM((CH, D), _f32) for _ in range(NBUF)],
        pltpu.VMEM_SHARED((N, D), _f32),
        [pltpu.SemaphoreType.DMA for _ in range(NCTX)],
        [pltpu.SemaphoreType.DMA for _ in range(NBUF)],
        [pltpu.SemaphoreType.DMA for _ in range(NBUF)],
    ],
)
def _sc_spmm(dst_h, src_h, wn_h, a_h, out_h,
             dstv, srcv, wnv, rows, acc, isem, gsem, ssem):
    cid = lax.axis_index("c")
    sid = lax.axis_index("s")
    wid = cid * 16 + sid
    rbase = wid * CPW

    def _z(i, c):
        for g in range(8):
            rows[0][i, pl.ds(g * 16, 16)] = jnp.zeros((16,), _f32)
        return c

    lax.fori_loop(0, CH, _z, 0)
    nbase = sid * 640

    @pl.when(sid < 15)
    def _():
        for k in range(5):
            pltpu.sync_copy(rows[0], acc.at[pl.ds(nbase + k * CH, CH)])

    @pl.when(sid == 15)
    def _():
        for k in range(3):
            pltpu.sync_copy(rows[0], acc.at[pl.ds(nbase + k * CH, CH)])
        pltpu.sync_copy(rows[0].at[pl.ds(0, 16)],
                        acc.at[pl.ds(nbase + 3 * CH, 16)])

    plsc.subcore_barrier()

    def _issue_idx(j, c):
        pltpu.async_copy(dst_h.at[rbase + j], dstv[c], isem[c])
        pltpu.async_copy(src_h.at[rbase + j], srcv[c], isem[c])
        pltpu.async_copy(wn_h.at[rbase + j], wnv[c], isem[c])

    def _wait_idx(c):
        pltpu.make_async_copy(dst_h.at[0], dstv[c], isem[c]).wait()
        pltpu.make_async_copy(src_h.at[0], srcv[c], isem[c]).wait()
        pltpu.make_async_copy(wn_h.at[0], wnv[c], isem[c]).wait()

    def _gather(b, c):
        pltpu.async_copy(a_h.at[dstv[c]], rows[b], gsem[b])

    def _wait_gather(b):
        pltpu.make_async_copy(a_h.at[pl.ds(0, CH)], rows[b], gsem[b]).wait()

    def _scatter(b, c):
        pltpu.async_copy(rows[b], acc.at[srcv[c]], ssem[b], add=True)

    def _wait_scatter(b):
        pltpu.make_async_copy(rows[b], acc.at[pl.ds(0, CH)], ssem[b]).wait()

    def _scale(b, c):
        def _sg(g, cc):
            wgrp = wnv[c][pl.ds(g * 16, 16)]
            for lane in range(16):
                wvec = jnp.full((16,), wgrp[lane], _f32)
                e = g * 16 + lane
                for cg in range(8):
                    rows[b][e, pl.ds(cg * 16, 16)] = (
                        rows[b][e, pl.ds(cg * 16, 16)] * wvec)
            return cc

        lax.fori_loop(0, CH // 16, _sg, 0)

    def _slot(j, m3, m4, do_sw, do_idx, do_g):
        _wait_gather(m3)
        _scale(m3, m4)
        _scatter(m3, m4)
        if do_sw:
            _wait_scatter((m3 + 2) % NBUF)
        if do_idx:
            _issue_idx(j + 3, (m4 + 3) % NCTX)
        if do_g:
            _wait_idx((m4 + 2) % NCTX)
            _gather((m3 + 2) % NBUF, (m4 + 2) % NCTX)

    for c in range(3):
        _issue_idx(c, c)
    _wait_idx(0)
    _gather(0, 0)
    _wait_idx(1)
    _gather(1, 1)

    _slot(0, 0, 0, False, True, True)

    def _steady(io, c):
        for k in range(12):
            j = 1 + io * 12 + k
            _slot(j, (1 + k) % NBUF, (1 + k) % NCTX, True, True, True)
        return c

    lax.fori_loop(0, 6, _steady, 0)

    for j in range(73, CPW):
        _slot(j, j % NBUF, j % NCTX, True, j + 3 < CPW, j + 2 < CPW)

    _wait_scatter((CPW - 1) % NBUF)
    plsc.subcore_barrier()

    @pl.when(sid < 15)
    def _():
        for k in range(5):
            pltpu.sync_copy(acc.at[pl.ds(nbase + k * CH, CH)],
                            out_h.at[cid, pl.ds(nbase + k * CH, CH)])

    @pl.when(sid == 15)
    def _():
        for k in range(3):
            pltpu.sync_copy(acc.at[pl.ds(nbase + k * CH, CH)],
                            out_h.at[cid, pl.ds(nbase + k * CH, CH)])
        pltpu.sync_copy(acc.at[pl.ds(nbase + 3 * CH, 16)],
                        out_h.at[cid, pl.ds(nbase + 3 * CH, 16)])


def _mm_relu_body(x_ref, w_ref, b_ref, o_ref):
    o_ref[...] = jnp.maximum(
        jnp.dot(x_ref[...], w_ref[...], preferred_element_type=_f32)
        + b_ref[...], 0.0)


def _mm_relu(x, w, b):
    return pl.pallas_call(
        _mm_relu_body,
        grid=(GRID,),
        in_specs=[
            pl.BlockSpec((R, D), lambda i: (i, 0)),
            pl.BlockSpec((D, D), lambda i: (0, 0)),
            pl.BlockSpec((1, D), lambda i: (0, 0)),
        ],
        out_specs=pl.BlockSpec((R, D), lambda i: (i, 0)),
        out_shape=jax.ShapeDtypeStruct((N, D), _f32),
    )(x, w, b)


def _stats_body(p0_ref, p1_ref, o_ref):
    i = pl.program_id(0)
    s = p0_ref[...] + p1_ref[...]
    s1 = jnp.sum(s, axis=0, keepdims=True)
    s2 = jnp.sum(s * s, axis=0, keepdims=True)
    st = jnp.concatenate([s1, s2, jnp.zeros((6, D), _f32)], axis=0)

    @pl.when(i == 0)
    def _():
        o_ref[...] = st

    @pl.when(i > 0)
    def _():
        o_ref[...] = o_ref[...] + st


def _stats(p0, p1):
    return pl.pallas_call(
        _stats_body,
        grid=(GRID,),
        in_specs=[
            pl.BlockSpec((R, D), lambda i: (i, 0)),
            pl.BlockSpec((R, D), lambda i: (i, 0)),
        ],
        out_specs=pl.BlockSpec((8, D), lambda i: (0, 0)),
        out_shape=jax.ShapeDtypeStruct((8, D), _f32),
    )(p0, p1)


def _gn_from_stats(s, st_ref, ga, gg, gb):
    m = st_ref[0:1, :] * (1.0 / N)
    ex2 = st_ref[1:2, :] * (1.0 / N)
    var = ex2 - (2.0 * ga - ga * ga) * m * m
    sub = s - ga * m
    return gg * sub * lax.rsqrt(var + EPS) + gb


def _apply0_body(p0_ref, p1_ref, st_ref, h0_ref, ga_ref, gg_ref, gb_ref,
                 cwa_ref, cwb_ref, cb_ref, h1_ref, st1_ref):
    i = pl.program_id(0)
    s = p0_ref[...] + p1_ref[...]
    y = _gn_from_stats(s, st_ref, ga_ref[...], gg_ref[...], gb_ref[...])
    h1 = (jnp.dot(y, cwa_ref[...], preferred_element_type=_f32)
          + jnp.dot(h0_ref[...], cwb_ref[...], preferred_element_type=_f32)
          + cb_ref[...])
    h1_ref[...] = h1
    s1 = jnp.sum(h1, axis=0, keepdims=True)
    s2 = jnp.sum(h1 * h1, axis=0, keepdims=True)
    st = jnp.concatenate([s1, s2, jnp.zeros((6, D), _f32)], axis=0)

    @pl.when(i == 0)
    def _():
        st1_ref[...] = st

    @pl.when(i > 0)
    def _():
        st1_ref[...] = st1_ref[...] + st


def _apply0(p0, p1, st, h0, ga, gg, gb, cwa, cwb, cb):
    return pl.pallas_call(
        _apply0_body,
        grid=(GRID,),
        in_specs=[
            pl.BlockSpec((R, D), lambda i: (i, 0)),
            pl.BlockSpec((R, D), lambda i: (i, 0)),
            pl.BlockSpec((8, D), lambda i: (0, 0)),
            pl.BlockSpec((R, D), lambda i: (i, 0)),
            pl.BlockSpec((1, D), lambda i: (0, 0)),
            pl.BlockSpec((1, D), lambda i: (0, 0)),
            pl.BlockSpec((1, D), lambda i: (0, 0)),
            pl.BlockSpec((D, D), lambda i: (0, 0)),
            pl.BlockSpec((D, D), lambda i: (0, 0)),
            pl.BlockSpec((1, D), lambda i: (0, 0)),
        ],
        out_specs=[
            pl.BlockSpec((R, D), lambda i: (i, 0)),
            pl.BlockSpec((8, D), lambda i: (0, 0)),
        ],
        out_shape=[
            jax.ShapeDtypeStruct((N, D), _f32),
            jax.ShapeDtypeStruct((8, D), _f32),
        ],
    )(p0, p1, st, h0, ga, gg, gb, cwa, cwb, cb)


def _mid_body(h1_ref, st_ref, ga_ref, gg_ref, gb_ref, tw_ref, tb_ref,
              h_ref, a1_ref):
    y = _gn_from_stats(h1_ref[...], st_ref, ga_ref[...], gg_ref[...],
                       gb_ref[...])
    h = jnp.maximum(y, 0.0)
    h_ref[...] = h
    a1_ref[...] = jnp.maximum(
        jnp.dot(h, tw_ref[...], preferred_element_type=_f32) + tb_ref[...],
        0.0)


def _mid(h1, st, ga, gg, gb, tw, tb):
    return pl.pallas_call(
        _mid_body,
        grid=(GRID,),
        in_specs=[
            pl.BlockSpec((R, D), lambda i: (i, 0)),
            pl.BlockSpec((8, D), lambda i: (0, 0)),
            pl.BlockSpec((1, D), lambda i: (0, 0)),
            pl.BlockSpec((1, D), lambda i: (0, 0)),
            pl.BlockSpec((1, D), lambda i: (0, 0)),
            pl.BlockSpec((D, D), lambda i: (0, 0)),
            pl.BlockSpec((1, D), lambda i: (0, 0)),
        ],
        out_specs=[
            pl.BlockSpec((R, D), lambda i: (i, 0)),
            pl.BlockSpec((R, D), lambda i: (i, 0)),
        ],
        out_shape=[
            jax.ShapeDtypeStruct((N, D), _f32),
            jax.ShapeDtypeStruct((N, D), _f32),
        ],
    )(h1, st, ga, gg, gb, tw, tb)


def _apply1_body(p0_ref, p1_ref, st_ref, h_ref, ga_ref, gg_ref, gb_ref,
                 cwa_ref, cwb_ref, cb_ref, o_ref):
    s = p0_ref[...] + p1_ref[...]
    y = _gn_from_stats(s, st_ref, ga_ref[...], gg_ref[...], gb_ref[...])
    o_ref[...] = (jnp.dot(y, cwa_ref[...], preferred_element_type=_f32)
                  + jnp.dot(h_ref[...], cwb_ref[...],
                            preferred_element_type=_f32)
                  + cb_ref[...])


def _apply1(p0, p1, st, h, ga, gg, gb, cwa, cwb, cb):
    return pl.pallas_call(
        _apply1_body,
        grid=(GRID,),
        in_specs=[
            pl.BlockSpec((R, D), lambda i: (i, 0)),
            pl.BlockSpec((R, D), lambda i: (i, 0)),
            pl.BlockSpec((8, D), lambda i: (0, 0)),
            pl.BlockSpec((R, D), lambda i: (i, 0)),
            pl.BlockSpec((1, D), lambda i: (0, 0)),
            pl.BlockSpec((1, D), lambda i: (0, 0)),
            pl.BlockSpec((1, D), lambda i: (0, 0)),
            pl.BlockSpec((D, D), lambda i: (0, 0)),
            pl.BlockSpec((D, D), lambda i: (0, 0)),
            pl.BlockSpec((1, D), lambda i: (0, 0)),
        ],
        out_specs=pl.BlockSpec((R, D), lambda i: (i, 0)),
        out_shape=jax.ShapeDtypeStruct((N, D), _f32),
    )(p0, p1, st, h, ga, gg, gb, cwa, cwb, cb)


def kernel(x, edge_index, edge_weight, emb,
           t_w0, t_b0, c_w0, c_b0, gnc_a0, gnc_g0, gnc_b0,
           t_w1, t_b1, c_w1, c_b1, gnc_a1, gnc_g1, gnc_b1,
           gn_a, gn_g, gn_b):
    pad_idx = jnp.arange(EP - E, dtype=jnp.int32) % N
    src = jnp.concatenate([edge_index[0], pad_idx]).reshape(EP // CH, CH)
    dst = jnp.concatenate([edge_index[1], pad_idx]).reshape(EP // CH, CH)
    ew = jnp.pad(edge_weight, (0, EP - E)).reshape(EP // CH, CH)
    xp = jnp.pad(x.astype(jnp.int32), (0, NP - N))

    wn, h0p, _ = _sc_prep(src, ew, xp, emb)
    h0 = h0p[:N]

    def r2(v):
        return v.reshape(1, D)

    a0 = _mm_relu(h0, t_w0, r2(t_b0))
    p = _sc_spmm(dst, src, wn, a0)
    p0, p1 = p[0], p[1]
    st0 = _stats(p0, p1)
    h1, st1 = _apply0(p0, p1, st0, h0,
                      r2(gnc_a0), r2(gnc_g0), r2(gnc_b0),
                      c_w0[:D], c_w0[D:], r2(c_b0))
    h, a1 = _mid(h1, st1, r2(gn_a), r2(gn_g), r2(gn_b), t_w1, r2(t_b1))
    q = _sc_spmm(dst, src, wn, a1)
    q0, q1 = q[0], q[1]
    st2 = _stats(q0, q1)
    out = _apply1(q0, q1, st2, h,
                  r2(gnc_a1), r2(gnc_g1), r2(gnc_b1),
                  c_w1[:D], c_w1[D:], r2(c_b1))
    return out


# speedup vs baseline: 15.0237x; 1.0548x over previous
import functools

import jax
import jax.numpy as jnp
from jax import lax
from jax.experimental import pallas as pl
from jax.experimental.pallas import tpu as pltpu
from jax.experimental.pallas import tpu_sc as plsc

N = 10000
NP = 10240
D = 128
E = 320000
EP = 327680
CH = 128
EPS = 1e-5
R = 1000
GRID = N // R

CPW = EP // 32 // CH
CPT = EP // 16 // CH

_f32 = jnp.float32
_mesh = plsc.VectorSubcoreMesh(core_axis_name="c", subcore_axis_name="s")


@functools.partial(
    pl.kernel,
    out_type=(
        jax.ShapeDtypeStruct((EP // CH, CH), _f32),
        jax.ShapeDtypeStruct((NP, D), _f32),
        jax.ShapeDtypeStruct((NP,), _f32),
    ),
    mesh=_mesh,
    scratch_types=[
        pltpu.VMEM((CH,), jnp.int32),
        pltpu.VMEM((CH, D), _f32),
        pltpu.VMEM((CPT, CH), jnp.int32),
        pltpu.VMEM((CPT, CH), _f32),
        pltpu.VMEM((CPT, CH), _f32),
        pltpu.VMEM((640,), _f32),
        pltpu.VMEM_SHARED((NP,), _f32),
        pltpu.SemaphoreType.DMA,
        pltpu.SemaphoreType.DMA,
    ],
)
def _sc_prep(src_h, ew_h, xidx_h, emb_h, wn_h, h0_h, ideg_h,
             idxv, rowv, srcm, ewm, idegm, degv, deg_acc, sem, dsem):
    cid = lax.axis_index("c")
    sid = lax.axis_index("s")

    @pl.when(cid == 1)
    def _():
        base = sid * 640
        for k in range(5):
            pltpu.sync_copy(xidx_h.at[pl.ds(base + k * CH, CH)], idxv)
            pltpu.async_copy(emb_h.at[idxv], rowv, sem).wait()
            pltpu.sync_copy(rowv, h0_h.at[pl.ds(base + k * CH, CH)])

    @pl.when(cid == 0)
    def _():
        pltpu.sync_copy(src_h.at[pl.ds(sid * CPT, CPT)], srcm)
        pltpu.sync_copy(ew_h.at[pl.ds(sid * CPT, CPT)], ewm)
        for k in range(40):
            degv[pl.ds(k * 16, 16)] = jnp.zeros((16,), _f32)
        pltpu.sync_copy(degv, deg_acc.at[pl.ds(sid * 640, 640)])
        plsc.subcore_barrier()

        def _deg_body(io, c):
            for k in range(8):
                j = io * 8 + k
                pltpu.async_copy(ewm.at[j], deg_acc.at[srcm.at[j]], dsem,
                                 add=True)
            for k in range(8):
                pltpu.make_async_copy(ewm.at[0], deg_acc.at[pl.ds(0, CH)],
                                      dsem).wait()
            return c

        lax.fori_loop(0, CPT // 8, _deg_body, 0)
        plsc.subcore_barrier()

        pltpu.sync_copy(deg_acc.at[pl.ds(sid * 640, 640)], degv)
        for k in range(40):
            dv = degv[pl.ds(k * 16, 16)]
            dv = jnp.where(dv < 0.5, dv + 1.0, dv)
            degv[pl.ds(k * 16, 16)] = 1.0 / dv
        pltpu.sync_copy(degv, ideg_h.at[pl.ds(sid * 640, 640)])
        plsc.subcore_barrier()

        def _ig(io, c):
            for k in range(8):
                j = io * 8 + k
                pltpu.async_copy(ideg_h.at[srcm.at[j]], idegm.at[j], dsem)
            for k in range(8):
                pltpu.make_async_copy(ideg_h.at[pl.ds(0, CH)], idegm.at[0],
                                      dsem).wait()
            return c

        lax.fori_loop(0, CPT // 8, _ig, 0)

        def _mul(j, c):
            for g in range(8):
                ewm[j, pl.ds(g * 16, 16)] = (
                    ewm[j, pl.ds(g * 16, 16)] * idegm[j, pl.ds(g * 16, 16)])
            return c

        lax.fori_loop(0, CPT, _mul, 0)
        pltpu.sync_copy(ewm, wn_h.at[pl.ds(sid * CPT, CPT)])


NBUF = 3
NCTX = 4


@functools.partial(
    pl.kernel,
    out_type=(
        jax.ShapeDtypeStruct((N, D), _f32),
        jax.ShapeDtypeStruct((N, D), _f32),
    ),
    mesh=_mesh,
    scratch_types=[
        [pltpu.VMEM((CH,), jnp.int32) for _ in range(NCTX)],
        [pltpu.VMEM((CH,), jnp.int32) for _ in range(NCTX)],
        [pltpu.VMEM((CH,), _f32) for _ in range(NCTX)],
        [pltpu.VMEM((CH, D), _f32) for _ in range(NBUF)],
        pltpu.VMEM_SHARED((N, D), _f32),
        [pltpu.SemaphoreType.DMA for _ in range(NCTX)],
        [pltpu.SemaphoreType.DMA for _ in range(NBUF)],
        [pltpu.SemaphoreType.DMA for _ in range(NBUF)],
    ],
)
def _sc_spmm(dst_h, src_h, wn_h, a_h, out0_h, out1_h,
             dstv, srcv, wnv, rows, acc, isem, gsem, ssem):
    cid = lax.axis_index("c")
    sid = lax.axis_index("s")
    wid = cid * 16 + sid
    rbase = wid * CPW

    def _z(i, c):
        for g in range(8):
            rows[0][i, pl.ds(g * 16, 16)] = jnp.zeros((16,), _f32)
        return c

    lax.fori_loop(0, CH, _z, 0)
    nbase = sid * 640

    @pl.when(sid < 15)
    def _():
        for k in range(5):
            pltpu.sync_copy(rows[0], acc.at[pl.ds(nbase + k * CH, CH)])

    @pl.when(sid == 15)
    def _():
        for k in range(3):
            pltpu.sync_copy(rows[0], acc.at[pl.ds(nbase + k * CH, CH)])
        pltpu.sync_copy(rows[0].at[pl.ds(0, 16)],
                        acc.at[pl.ds(nbase + 3 * CH, 16)])

    plsc.subcore_barrier()

    def _issue_idx(j, c):
        pltpu.async_copy(dst_h.at[rbase + j], dstv[c], isem[c])
        pltpu.async_copy(src_h.at[rbase + j], srcv[c], isem[c])
        pltpu.async_copy(wn_h.at[rbase + j], wnv[c], isem[c])

    def _wait_idx(c):
        pltpu.make_async_copy(dst_h.at[0], dstv[c], isem[c]).wait()
        pltpu.make_async_copy(src_h.at[0], srcv[c], isem[c]).wait()
        pltpu.make_async_copy(wn_h.at[0], wnv[c], isem[c]).wait()

    def _gather(b, c):
        pltpu.async_copy(a_h.at[dstv[c]], rows[b], gsem[b])

    def _wait_gather(b):
        pltpu.make_async_copy(a_h.at[pl.ds(0, CH)], rows[b], gsem[b]).wait()

    def _scatter(b, c):
        pltpu.async_copy(rows[b], acc.at[srcv[c]], ssem[b], add=True)

    def _wait_scatter(b):
        pltpu.make_async_copy(rows[b], acc.at[pl.ds(0, CH)], ssem[b]).wait()

    def _scale(b, c):
        def _sg(g, cc):
            wgrp = wnv[c][pl.ds(g * 16, 16)]
            for lane in range(16):
                wvec = jnp.full((16,), wgrp[lane], _f32)
                e = g * 16 + lane
                for cg in range(8):
                    rows[b][e, pl.ds(cg * 16, 16)] = (
                        rows[b][e, pl.ds(cg * 16, 16)] * wvec)
            return cc

        lax.fori_loop(0, CH // 16, _sg, 0)

    def _slot(j, m3, m4, do_sw, do_idx, do_g):
        _wait_gather(m3)
        _scale(m3, m4)
        _scatter(m3, m4)
        if do_sw:
            _wait_scatter((m3 + 2) % NBUF)
        if do_idx:
            _issue_idx(j + 3, (m4 + 3) % NCTX)
        if do_g:
            _wait_idx((m4 + 2) % NCTX)
            _gather((m3 + 2) % NBUF, (m4 + 2) % NCTX)

    for c in range(3):
        _issue_idx(c, c)
    _wait_idx(0)
    _gather(0, 0)
    _wait_idx(1)
    _gather(1, 1)

    _slot(0, 0, 0, False, True, True)

    def _steady(io, c):
        for k in range(12):
            j = 1 + io * 12 + k
            _slot(j, (1 + k) % NBUF, (1 + k) % NCTX, True, True, True)
        return c

    lax.fori_loop(0, 6, _steady, 0)

    for j in range(73, CPW):
        _slot(j, j % NBUF, j % NCTX, True, j + 3 < CPW, j + 2 < CPW)

    _wait_scatter((CPW - 1) % NBUF)
    plsc.subcore_barrier()

    def _dump(oh):
        @pl.when(sid < 15)
        def _():
            for k in range(5):
                pltpu.sync_copy(acc.at[pl.ds(nbase + k * CH, CH)],
                                oh.at[pl.ds(nbase + k * CH, CH)])

        @pl.when(sid == 15)
        def _():
            for k in range(3):
                pltpu.sync_copy(acc.at[pl.ds(nbase + k * CH, CH)],
                                oh.at[pl.ds(nbase + k * CH, CH)])
            pltpu.sync_copy(acc.at[pl.ds(nbase + 3 * CH, 16)],
                            oh.at[pl.ds(nbase + 3 * CH, 16)])

    @pl.when(cid == 0)
    def _():
        _dump(out0_h)

    @pl.when(cid == 1)
    def _():
        _dump(out1_h)


def _mm_relu_body(x_ref, w_ref, b_ref, o_ref):
    o_ref[...] = jnp.maximum(
        jnp.dot(x_ref[...], w_ref[...], preferred_element_type=_f32)
        + b_ref[...], 0.0)


def _mm_relu(x, w, b):
    rp = NP // 10
    return pl.pallas_call(
        _mm_relu_body,
        grid=(10,),
        in_specs=[
            pl.BlockSpec((rp, D), lambda i: (i, 0)),
            pl.BlockSpec((D, D), lambda i: (0, 0)),
            pl.BlockSpec((1, D), lambda i: (0, 0)),
        ],
        out_specs=pl.BlockSpec((rp, D), lambda i: (i, 0)),
        out_shape=jax.ShapeDtypeStruct((NP, D), _f32),
    )(x, w, b)


def _stats_body(p0_ref, p1_ref, o_ref):
    i = pl.program_id(0)
    s = p0_ref[...] + p1_ref[...]
    s1 = jnp.sum(s, axis=0, keepdims=True)
    s2 = jnp.sum(s * s, axis=0, keepdims=True)
    st = jnp.concatenate([s1, s2, jnp.zeros((6, D), _f32)], axis=0)

    @pl.when(i == 0)
    def _():
        o_ref[...] = st

    @pl.when(i > 0)
    def _():
        o_ref[...] = o_ref[...] + st


def _stats(p0, p1):
    return pl.pallas_call(
        _stats_body,
        grid=(GRID,),
        in_specs=[
            pl.BlockSpec((R, D), lambda i: (i, 0)),
            pl.BlockSpec((R, D), lambda i: (i, 0)),
        ],
        out_specs=pl.BlockSpec((8, D), lambda i: (0, 0)),
        out_shape=jax.ShapeDtypeStruct((8, D), _f32),
    )(p0, p1)


def _gn_from_stats(s, st_ref, ga, gg, gb):
    m = st_ref[0:1, :] * (1.0 / N)
    ex2 = st_ref[1:2, :] * (1.0 / N)
    var = ex2 - (2.0 * ga - ga * ga) * m * m
    sub = s - ga * m
    return gg * sub * lax.rsqrt(var + EPS) + gb


def _apply0_body(p0_ref, p1_ref, st_ref, h0_ref, ga_ref, gg_ref, gb_ref,
                 cwa_ref, cwb_ref, cb_ref, h1_ref, st1_ref):
    i = pl.program_id(0)
    s = p0_ref[...] + p1_ref[...]
    y = _gn_from_stats(s, st_ref, ga_ref[...], gg_ref[...], gb_ref[...])
    h1 = (jnp.dot(y, cwa_ref[...], preferred_element_type=_f32)
          + jnp.dot(h0_ref[...], cwb_ref[...], preferred_element_type=_f32)
          + cb_ref[...])
    h1_ref[...] = h1
    s1 = jnp.sum(h1, axis=0, keepdims=True)
    s2 = jnp.sum(h1 * h1, axis=0, keepdims=True)
    st = jnp.concatenate([s1, s2, jnp.zeros((6, D), _f32)], axis=0)

    @pl.when(i == 0)
    def _():
        st1_ref[...] = st

    @pl.when(i > 0)
    def _():
        st1_ref[...] = st1_ref[...] + st


def _apply0(p0, p1, st, h0, ga, gg, gb, cwa, cwb, cb):
    return pl.pallas_call(
        _apply0_body,
        grid=(GRID,),
        in_specs=[
            pl.BlockSpec((R, D), lambda i: (i, 0)),
            pl.BlockSpec((R, D), lambda i: (i, 0)),
            pl.BlockSpec((8, D), lambda i: (0, 0)),
            pl.BlockSpec((R, D), lambda i: (i, 0)),
            pl.BlockSpec((1, D), lambda i: (0, 0)),
            pl.BlockSpec((1, D), lambda i: (0, 0)),
            pl.BlockSpec((1, D), lambda i: (0, 0)),
            pl.BlockSpec((D, D), lambda i: (0, 0)),
            pl.BlockSpec((D, D), lambda i: (0, 0)),
            pl.BlockSpec((1, D), lambda i: (0, 0)),
        ],
        out_specs=[
            pl.BlockSpec((R, D), lambda i: (i, 0)),
            pl.BlockSpec((8, D), lambda i: (0, 0)),
        ],
        out_shape=[
            jax.ShapeDtypeStruct((N, D), _f32),
            jax.ShapeDtypeStruct((8, D), _f32),
        ],
    )(p0, p1, st, h0, ga, gg, gb, cwa, cwb, cb)


def _mid_body(h1_ref, st_ref, ga_ref, gg_ref, gb_ref, tw_ref, tb_ref,
              h_ref, a1_ref):
    y = _gn_from_stats(h1_ref[...], st_ref, ga_ref[...], gg_ref[...],
                       gb_ref[...])
    h = jnp.maximum(y, 0.0)
    h_ref[...] = h
    a1_ref[...] = jnp.maximum(
        jnp.dot(h, tw_ref[...], preferred_element_type=_f32) + tb_ref[...],
        0.0)


def _mid(h1, st, ga, gg, gb, tw, tb):
    return pl.pallas_call(
        _mid_body,
        grid=(GRID,),
        in_specs=[
            pl.BlockSpec((R, D), lambda i: (i, 0)),
            pl.BlockSpec((8, D), lambda i: (0, 0)),
            pl.BlockSpec((1, D), lambda i: (0, 0)),
            pl.BlockSpec((1, D), lambda i: (0, 0)),
            pl.BlockSpec((1, D), lambda i: (0, 0)),
            pl.BlockSpec((D, D), lambda i: (0, 0)),
            pl.BlockSpec((1, D), lambda i: (0, 0)),
        ],
        out_specs=[
            pl.BlockSpec((R, D), lambda i: (i, 0)),
            pl.BlockSpec((R, D), lambda i: (i, 0)),
        ],
        out_shape=[
            jax.ShapeDtypeStruct((N, D), _f32),
            jax.ShapeDtypeStruct((N, D), _f32),
        ],
    )(h1, st, ga, gg, gb, tw, tb)


def _apply1_body(p0_ref, p1_ref, st_ref, h_ref, ga_ref, gg_ref, gb_ref,
                 cwa_ref, cwb_ref, cb_ref, o_ref):
    s = p0_ref[...] + p1_ref[...]
    y = _gn_from_stats(s, st_ref, ga_ref[...], gg_ref[...], gb_ref[...])
    o_ref[...] = (jnp.dot(y, cwa_ref[...], preferred_element_type=_f32)
                  + jnp.dot(h_ref[...], cwb_ref[...],
                            preferred_element_type=_f32)
                  + cb_ref[...])


def _apply1(p0, p1, st, h, ga, gg, gb, cwa, cwb, cb):
    return pl.pallas_call(
        _apply1_body,
        grid=(GRID,),
        in_specs=[
            pl.BlockSpec((R, D), lambda i: (i, 0)),
            pl.BlockSpec((R, D), lambda i: (i, 0)),
            pl.BlockSpec((8, D), lambda i: (0, 0)),
            pl.BlockSpec((R, D), lambda i: (i, 0)),
            pl.BlockSpec((1, D), lambda i: (0, 0)),
            pl.BlockSpec((1, D), lambda i: (0, 0)),
            pl.BlockSpec((1, D), lambda i: (0, 0)),
            pl.BlockSpec((D, D), lambda i: (0, 0)),
            pl.BlockSpec((D, D), lambda i: (0, 0)),
            pl.BlockSpec((1, D), lambda i: (0, 0)),
        ],
        out_specs=pl.BlockSpec((R, D), lambda i: (i, 0)),
        out_shape=jax.ShapeDtypeStruct((N, D), _f32),
    )(p0, p1, st, h, ga, gg, gb, cwa, cwb, cb)


def kernel(x, edge_index, edge_weight, emb,
           t_w0, t_b0, c_w0, c_b0, gnc_a0, gnc_g0, gnc_b0,
           t_w1, t_b1, c_w1, c_b1, gnc_a1, gnc_g1, gnc_b1,
           gn_a, gn_g, gn_b):
    pad_idx = jnp.arange(EP - E, dtype=jnp.int32) % N
    src = jnp.concatenate([edge_index[0], pad_idx]).reshape(EP // CH, CH)
    dst = jnp.concatenate([edge_index[1], pad_idx]).reshape(EP // CH, CH)
    ew = jnp.pad(edge_weight, (0, EP - E)).reshape(EP // CH, CH)
    xp = jnp.pad(x.astype(jnp.int32), (0, NP - N))

    wn, h0p, _ = _sc_prep(src, ew, xp, emb)

    def r2(v):
        return v.reshape(1, D)

    a0 = _mm_relu(h0p, t_w0, r2(t_b0))
    p0, p1 = _sc_spmm(dst, src, wn, a0)
    st0 = _stats(p0, p1)
    h1, st1 = _apply0(p0, p1, st0, h0p,
                      r2(gnc_a0), r2(gnc_g0), r2(gnc_b0),
                      c_w0[:D], c_w0[D:], r2(c_b0))
    h, a1 = _mid(h1, st1, r2(gn_a), r2(gn_g), r2(gn_b), t_w1, r2(t_b1))
    q0, q1 = _sc_spmm(dst, src, wn, a1)
    st2 = _stats(q0, q1)
    out = _apply1(q0, q1, st2, h,
                  r2(gnc_a1), r2(gnc_g1), r2(gnc_b1),
                  c_w1[:D], c_w1[D:], r2(c_b1))
    return out
